# Initial kernel scaffold; baseline (speedup 1.0000x reference)
import jax
import jax.numpy as jnp
from jax.experimental import pallas as pl


def kernel(captions, caption_lengths, logs, idall, dfall, ix, emb_table, W_out, b_out):
    raise NotImplementedError("write your pallas kernel here")


# speedup vs baseline: 17.2583x; 17.2583x over previous
import functools

import jax
import jax.numpy as jnp
from jax import lax
from jax.experimental import pallas as pl
from jax.experimental.pallas import tpu as pltpu
from jax.experimental.pallas import tpu_sc as plsc

B, BEAM, S, V, D = 4096, 8, 20, 100000, 128
N = B * BEAM
NW = 32
NPW = N // NW
ROW_TILE = 1024
VP = 100352

_mesh = plsc.VectorSubcoreMesh(core_axis_name="c", subcore_axis_name="s")


def _project_body(emb_ref, w_ref, bs_ref, out_ref):
    out_ref[...] = jnp.sum(emb_ref[...] * w_ref[...], axis=1) + bs_ref[0, 0]


def _project(emb_table, w, b_over_s):
    return pl.pallas_call(
        _project_body,
        grid=(VP // ROW_TILE,),
        in_specs=[
            pl.BlockSpec((ROW_TILE, D), lambda i: (i, 0)),
            pl.BlockSpec((1, D), lambda i: (0, 0)),
            pl.BlockSpec((1, 1), lambda i: (0, 0)),
        ],
        out_specs=pl.BlockSpec((ROW_TILE,), lambda i: (i,)),
        out_shape=jax.ShapeDtypeStruct((VP,), jnp.float32),
    )(emb_table, w, b_over_s)


@functools.partial(
    pl.kernel,
    mesh=_mesh,
    compiler_params=pltpu.CompilerParams(
        use_tc_tiling_on_sc=False, needs_layout_passes=False
    ),
    out_type=jax.ShapeDtypeStruct((NW, NPW), jnp.float32),
    scratch_types=[
        pltpu.VMEM((VP,), jnp.float32),
        pltpu.VMEM((S, NPW), jnp.int32),
        pltpu.VMEM((NPW,), jnp.int32),
        pltpu.VMEM((NPW,), jnp.int32),
        pltpu.VMEM((NPW,), jnp.int32),
        pltpu.VMEM((NPW,), jnp.float32),
        pltpu.VMEM((NPW,), jnp.float32),
    ],
)
def _sc_scores(p_hbm, cap_hbm, len_hbm, ida_hbm, ixr_hbm, df_hbm, out_hbm,
               p_v, cap_v, len_v, ida_v, ixr_v, df_v, sc_v):
    wid = lax.axis_index("s") * 2 + lax.axis_index("c")
    pltpu.sync_copy(p_hbm, p_v)
    pltpu.sync_copy(cap_hbm.at[wid], cap_v)
    pltpu.sync_copy(len_hbm.at[wid], len_v)
    pltpu.sync_copy(ida_hbm.at[wid], ida_v)
    pltpu.sync_copy(ixr_hbm.at[wid], ixr_v)
    pltpu.sync_copy(df_hbm.at[wid], df_v)

    zero_i = jnp.zeros((16,), jnp.int32)
    zero_f = jnp.zeros((16,), jnp.float32)

    def body(v, carry):
        o = v * 16
        lv = len_v[pl.ds(o, 16)]
        acc = zero_f
        for s in range(S):
            idx = cap_v[s, pl.ds(o, 16)]
            valid = lv > jnp.full((16,), s + 1, jnp.int32)
            acc = acc + plsc.load_gather(p_v, [jnp.where(valid, idx, zero_i)])
        eq = ida_v[pl.ds(o, 16)] == ixr_v[pl.ds(o, 16)]
        idf = jnp.where(eq, df_v[pl.ds(o, 16)], zero_f)
        sc_v[pl.ds(o, 16)] = acc * idf
        return carry

    lax.fori_loop(0, NPW // 16, body, 0)
    pltpu.sync_copy(sc_v, out_hbm.at[wid])


def _lsm_body(x_ref, o_ref):
    x = x_ref[...]
    m = jnp.max(x, axis=1, keepdims=True)
    lse = jnp.log(jnp.sum(jnp.exp(x - m), axis=1, keepdims=True)) + m
    o_ref[...] = x - lse


def _log_softmax(scores):
    return pl.pallas_call(
        _lsm_body,
        out_shape=jax.ShapeDtypeStruct((B, BEAM), jnp.float32),
    )(scores)


def kernel(captions, caption_lengths, logs, idall, dfall, ix, emb_table, W_out, b_out):
    p = _project(emb_table, W_out, (b_out / S).reshape(1, 1))
    cap3 = (
        captions.reshape(N, S).T.reshape(S, NW, NPW).transpose(1, 0, 2)
    )
    len2 = caption_lengths.reshape(NW, NPW)
    ida2 = idall.reshape(NW, NPW)
    ixr2 = jnp.broadcast_to(ix[:, None], (B, BEAM)).reshape(NW, NPW)
    df2 = dfall.reshape(NW, NPW)
    scores = _sc_scores(p, cap3, len2, ida2, ixr2, df2).reshape(B, BEAM)
    return _log_softmax(scores)
